# baseline (speedup 1.0000x reference)
import jax
import jax.numpy as jnp
from jax import lax
from jax.experimental import pallas as pl
from jax.experimental.pallas import tpu as pltpu
from jax.experimental.pallas import tpu_sc as plsc

N_ENT = 1000000
RANK = 32
BATCH = 4096

_info = plsc.get_sparse_core_info()
_NC, _NS, _L = _info.num_cores, _info.num_subcores, _info.num_lanes
_NW = _NC * _NS
_BPW = BATCH // _NW
_GROUPS = _BPW // _L


def _body(hq_hbm, rq_hbm, tq_hbm, entity_hbm, rel_hbm, bh_hbm, bt_hbm,
          pred_out, fh_out, fr_out, ft_out,
          hv, rv, tv, head_v, rel_v, rhs_v, bhv, btv, sq_v, pred_v,
          gsem, wsem):
    wid = lax.axis_index("s") * _NC + lax.axis_index("c")
    base = wid * _BPW

    pltpu.sync_copy(hq_hbm.at[pl.ds(base, _BPW)], hv)
    pltpu.sync_copy(rq_hbm.at[pl.ds(base, _BPW)], rv)
    pltpu.sync_copy(tq_hbm.at[pl.ds(base, _BPW)], tv)

    g1 = pltpu.async_copy(entity_hbm.at[hv], head_v, gsem)
    g2 = pltpu.async_copy(rel_hbm.at[rv], rel_v, gsem)
    g3 = pltpu.async_copy(entity_hbm.at[tv], rhs_v, gsem)
    g4 = pltpu.async_copy(bh_hbm.at[hv], bhv, gsem)
    g5 = pltpu.async_copy(bt_hbm.at[tv], btv, gsem)
    g1.wait(); g2.wait(); g3.wait(); g4.wait(); g5.wait()

    w1 = pltpu.async_copy(head_v, fh_out.at[pl.ds(base, _BPW)], wsem)
    w2 = pltpu.async_copy(rel_v, fr_out.at[pl.ds(base, _BPW)], wsem)
    w3 = pltpu.async_copy(rhs_v, ft_out.at[pl.ds(base, _BPW)], wsem)

    for i in range(_BPW):
        e0 = (head_v[i, pl.ds(0, _L)] + rel_v[i, pl.ds(0, _L)]
              ) - rhs_v[i, pl.ds(0, _L)]
        e1 = (head_v[i, pl.ds(_L, _L)] + rel_v[i, pl.ds(_L, _L)]
              ) - rhs_v[i, pl.ds(_L, _L)]
        sq_v[pl.ds(i * _L, _L)] = e0 * e0 + e1 * e1

    lane = jnp.arange(_L, dtype=jnp.int32)
    for g in range(_GROUPS):
        row0 = (g * _L + lane) * _L
        acc = jnp.zeros((_L,), dtype=jnp.float32)
        for j in range(_L):
            acc = acc + plsc.load_gather(sq_v, [row0 + j])
        bias = bhv[pl.ds(g * _L, _L)] + btv[pl.ds(g * _L, _L)]
        pred_v[pl.ds(g * _L, _L)] = bias - acc

    pltpu.sync_copy(pred_v, pred_out.at[pl.ds(base, _BPW)])
    w1.wait(); w2.wait(); w3.wait()


@jax.jit
def kernel(queries, entity, rel, bh, bt):
    mesh = plsc.VectorSubcoreMesh(core_axis_name="c", subcore_axis_name="s")
    f32 = jnp.float32
    run = pl.kernel(
        _body,
        mesh=mesh,
        compiler_params=pltpu.CompilerParams(
            needs_layout_passes=False, use_tc_tiling_on_sc=False),
        out_type=[
            jax.ShapeDtypeStruct((BATCH,), f32),
            jax.ShapeDtypeStruct((BATCH, RANK), f32),
            jax.ShapeDtypeStruct((BATCH, RANK), f32),
            jax.ShapeDtypeStruct((BATCH, RANK), f32),
        ],
        scratch_types=[
            pltpu.VMEM((_BPW,), jnp.int32),
            pltpu.VMEM((_BPW,), jnp.int32),
            pltpu.VMEM((_BPW,), jnp.int32),
            pltpu.VMEM((_BPW, RANK), f32),
            pltpu.VMEM((_BPW, RANK), f32),
            pltpu.VMEM((_BPW, RANK), f32),
            pltpu.VMEM((_BPW,), f32),
            pltpu.VMEM((_BPW,), f32),
            pltpu.VMEM((_BPW * _L,), f32),
            pltpu.VMEM((_BPW,), f32),
            pltpu.SemaphoreType.DMA,
            pltpu.SemaphoreType.DMA,
        ],
    )
    queries = queries.astype(jnp.int32)
    hq = queries[:, 0]
    rq = queries[:, 1]
    tq = queries[:, 2]
    pred, fh, fr, ft = run(hq, rq, tq, entity, rel,
                           bh.reshape(N_ENT), bt.reshape(N_ENT))
    return (pred.reshape(BATCH, 1), fh, fr, ft)
